# baseline (speedup 1.0000x reference)
import functools

import jax
import jax.numpy as jnp
from jax import lax
from jax.experimental import pallas as pl
from jax.experimental.pallas import tpu as pltpu
from jax.experimental.pallas import tpu_sc as plsc

N = 10000
D = 128
H = 128
E = 320000

NC = 2
NS = 16
NW = NC * NS
CHUNK = 128
SUPER = 16
K0 = 10
K1 = 0
TOTAL_SUPER = NS * (K0 + K1)
EPAD = TOTAL_SUPER * SUPER * CHUNK
ROW_STRIDE = 624
ROW_SPAN = 640


def _spmm_body(h_hbm, row_hbm, col_hbm, val_hbm, out_hbm,
               col_a, col_b, row_a, row_b, val_a, val_b,
               msg_a, msg_b, acc_sh, sem_a, sem_b, sem_ma, sem_mb):
    cid = lax.axis_index("c")
    sid = lax.axis_index("s")
    wid = cid * NS + sid

    cols = (col_a, col_b)
    rows = (row_a, row_b)
    vals = (val_a, val_b)
    sem_m = (sem_ma, sem_mb)

    zeros16 = jnp.zeros((16,), jnp.float32)

    def _zero_row(r, _):
        for j in range(8):
            msg_a[r, pl.ds(j * 16, 16)] = zeros16
        return _

    with jax.named_scope("zero_acc"):
        lax.fori_loop(0, CHUNK, _zero_row, None)
        r0 = sid * ROW_STRIDE
        for part in range(ROW_SPAN // CHUNK):
            pltpu.sync_copy(msg_a.at[pl.ds(0, CHUNK)],
                            acc_sh.at[pl.ds(r0 + part * CHUNK, CHUNK)])
        plsc.subcore_barrier()

    def _meta_start(g, b):
        pltpu.async_copy(col_hbm.at[g], cols[b], sem_m[b])
        pltpu.async_copy(row_hbm.at[g], rows[b], sem_m[b])
        pltpu.async_copy(val_hbm.at[g], vals[b], sem_m[b])

    def _meta_wait(g, b):
        pltpu.make_async_copy(col_hbm.at[g], cols[b], sem_m[b]).wait()
        pltpu.make_async_copy(row_hbm.at[g], rows[b], sem_m[b]).wait()
        pltpu.make_async_copy(val_hbm.at[g], vals[b], sem_m[b]).wait()

    def _scale_chunk(val_s, j, msg):
        def _scale(g, __):
            val16 = val_s[j, pl.ds(g * 16, 16)]
            for k in range(16):
                vb = _lane_bcast(val16, k)
                e = g * 16 + k
                for jj in range(8):
                    sl = pl.ds(jj * 16, 16)
                    msg[e, sl] = msg[e, sl] * vb
            return __

        lax.fori_loop(0, CHUNK // 16, _scale, None)

    def _run_super(b):
        col_s, row_s, val_s = cols[b], rows[b], vals[b]
        pltpu.async_copy(h_hbm.at[col_s.at[0]], msg_a, sem_a)
        pltpu.async_copy(h_hbm.at[col_s.at[1]], msg_b, sem_b)

        def _chunk_pair(t, _):
            j0 = 2 * t

            pltpu.make_async_copy(h_hbm.at[col_s.at[j0]], msg_a,
                                  sem_a).wait()
            _scale_chunk(val_s, j0, msg_a)
            pltpu.sync_copy(msg_a, acc_sh.at[row_s.at[j0]], add=True)

            @pl.when(t < SUPER // 2 - 1)
            def _():
                pltpu.async_copy(h_hbm.at[col_s.at[j0 + 2]], msg_a,
                                 sem_a)

            pltpu.make_async_copy(h_hbm.at[col_s.at[j0 + 1]], msg_b,
                                  sem_b).wait()
            _scale_chunk(val_s, j0 + 1, msg_b)
            pltpu.sync_copy(msg_b, acc_sh.at[row_s.at[j0 + 1]],
                            add=True)

            @pl.when(t < SUPER // 2 - 1)
            def _():
                pltpu.async_copy(h_hbm.at[col_s.at[j0 + 3]], msg_b,
                                 sem_b)

            return _

        lax.fori_loop(0, SUPER // 2, _chunk_pair, None)

    def _run_supers(base_g, npairs):
        @pl.when(npairs > 0)
        def _():
            _meta_start(base_g, 0)
            _meta_start(base_g + 1, 1)
            _meta_wait(base_g, 0)

            def _super_pair(t, _):
                g0 = base_g + 2 * t
                _run_super(0)

                @pl.when(t < npairs - 1)
                def _():
                    _meta_start(g0 + 2, 0)

                _meta_wait(g0 + 1, 1)
                _run_super(1)

                @pl.when(t < npairs - 1)
                def _():
                    _meta_start(g0 + 3, 1)
                    _meta_wait(g0 + 2, 0)

                return _

            lax.fori_loop(0, npairs, _super_pair, None)

    with jax.named_scope("edge_loop"):
        kk = jnp.where(cid == 0, K0, K1)
        base = jnp.where(cid == 0, sid * K0, NS * K0 + sid * K1)
        _run_supers(base, kk // 2)
        plsc.subcore_barrier()

    with jax.named_scope("copy_out"):
        pltpu.sync_copy(acc_sh.at[pl.ds(r0, ROW_SPAN)],
                        out_hbm.at[cid, pl.ds(r0, ROW_SPAN)])


_spmm = pl.kernel(
    _spmm_body,
    out_type=jax.ShapeDtypeStruct((NC, N, H), jnp.float32),
    mesh=plsc.VectorSubcoreMesh(core_axis_name="c", subcore_axis_name="s"),
    scratch_types=[
        pltpu.VMEM((SUPER, CHUNK), jnp.int32),
        pltpu.VMEM((SUPER, CHUNK), jnp.int32),
        pltpu.VMEM((SUPER, CHUNK), jnp.int32),
        pltpu.VMEM((SUPER, CHUNK), jnp.int32),
        pltpu.VMEM((SUPER, CHUNK), jnp.float32),
        pltpu.VMEM((SUPER, CHUNK), jnp.float32),
        pltpu.VMEM((CHUNK, H), jnp.float32),
        pltpu.VMEM((CHUNK, H), jnp.float32),
        pltpu.VMEM_SHARED((N, H), jnp.float32),
        pltpu.SemaphoreType.DMA,
        pltpu.SemaphoreType.DMA,
        pltpu.SemaphoreType.DMA,
        pltpu.SemaphoreType.DMA,
    ],
)


_GATHER_DNUMS = lax.GatherDimensionNumbers(
    offset_dims=(), collapsed_slice_dims=(0,), start_index_map=(0,))


def _lane_bcast(v16, k):
    return lax.gather(v16, jnp.full((16, 1), k, jnp.int32), _GATHER_DNUMS,
                      slice_sizes=(1,),
                      mode=lax.GatherScatterMode.PROMISE_IN_BOUNDS)


_SQRT_HALF = 0.7071067811865476


def _gelu(s):
    return 0.5 * s * (1.0 + lax.erf(s * _SQRT_HALF))


def _linear1_body(x_ref, w_ref, b_ref, o_ref):
    o_ref[...] = (
        jnp.dot(x_ref[...], w_ref[...], preferred_element_type=jnp.float32,
                precision=lax.Precision.HIGHEST)
        + b_ref[...]
    )


def _mid_body(p0_ref, p1_ref, w_ref, b_ref, o_ref):
    g = _gelu(p0_ref[...] + p1_ref[...])
    o_ref[...] = (
        jnp.dot(g, w_ref[...], preferred_element_type=jnp.float32,
                precision=lax.Precision.HIGHEST)
        + b_ref[...]
    )


def _final_body(p0_ref, p1_ref, x_ref, g_ref, b_ref, o_ref):
    h = _gelu(p0_ref[...] + p1_ref[...]) + x_ref[...]
    mean = jnp.mean(h, axis=-1, keepdims=True)
    var = jnp.mean((h - mean) ** 2, axis=-1, keepdims=True)
    o_ref[...] = (h - mean) * lax.rsqrt(var + 1e-5) * g_ref[...] + b_ref[...]


_BLK = 1000
_GRID = N // _BLK

_row_spec = pl.BlockSpec((_BLK, H), lambda i: (i, 0))
_full_spec = pl.BlockSpec((H, H), lambda i: (0, 0))
_vec_spec = pl.BlockSpec((1, H), lambda i: (0, 0))

_linear1 = pl.pallas_call(
    _linear1_body,
    grid=(_GRID,),
    in_specs=[_row_spec, _full_spec, _vec_spec],
    out_specs=_row_spec,
    out_shape=jax.ShapeDtypeStruct((N, H), jnp.float32),
)

_mid = pl.pallas_call(
    _mid_body,
    grid=(_GRID,),
    in_specs=[_row_spec, _row_spec, _full_spec, _vec_spec],
    out_specs=_row_spec,
    out_shape=jax.ShapeDtypeStruct((N, H), jnp.float32),
)

_final = pl.pallas_call(
    _final_body,
    grid=(_GRID,),
    in_specs=[_row_spec, _row_spec, _row_spec, _vec_spec, _vec_spec],
    out_specs=_row_spec,
    out_shape=jax.ShapeDtypeStruct((N, H), jnp.float32),
)


@jax.jit
def kernel(x, adj_indices, adj_values, W1, b1, W2, b2, ln_gamma, ln_beta):
    pad = EPAD - E
    row = jnp.concatenate([adj_indices[0], jnp.zeros((pad,), jnp.int32)])
    col = jnp.concatenate([adj_indices[1], jnp.zeros((pad,), jnp.int32)])
    val = jnp.concatenate([adj_values, jnp.zeros((pad,), jnp.float32)])
    row = row.reshape(TOTAL_SUPER, SUPER, CHUNK)
    col = col.reshape(TOTAL_SUPER, SUPER, CHUNK)
    val = val.reshape(TOTAL_SUPER, SUPER, CHUNK)

    b1r = b1.reshape(1, H)
    b2r = b2.reshape(1, H)
    gr = ln_gamma.reshape(1, H)
    br = ln_beta.reshape(1, H)

    h1 = _linear1(x, W1.T, b1r)
    p1 = _spmm(h1, row, col, val)
    h2 = _mid(p1[0], p1[1], W2.T, b2r)
    p2 = _spmm(h2, row, col, val)
    return _final(p2[0], p2[1], x[:, :H], gr, br)


# speedup vs baseline: 1.3694x; 1.3694x over previous
import functools

import jax
import jax.numpy as jnp
from jax import lax
from jax.experimental import pallas as pl
from jax.experimental.pallas import tpu as pltpu
from jax.experimental.pallas import tpu_sc as plsc

N = 10000
D = 128
H = 128
E = 320000

NC = 2
NS = 16
NW = NC * NS
CHUNK = 128
SUPER = 16
K0 = 8
K1 = 2
TOTAL_SUPER = NS * (K0 + K1)
EPAD = TOTAL_SUPER * SUPER * CHUNK
ROW_STRIDE = 624
ROW_SPAN = 640


def _spmm_body(h_hbm, hc_hbm, row_hbm, col_hbm, val_hbm, out_hbm,
               col_a, col_b, row_a, row_b, val_a, val_b,
               msg_a, msg_b, acc_sh, sem_a, sem_b, sem_ma, sem_mb):
    cid = lax.axis_index("c")
    sid = lax.axis_index("s")
    wid = cid * NS + sid

    cols = (col_a, col_b)
    rows = (row_a, row_b)
    vals = (val_a, val_b)
    sem_m = (sem_ma, sem_mb)

    zeros16 = jnp.zeros((16,), jnp.float32)

    def _zero_row(r, _):
        for j in range(8):
            msg_a[r, pl.ds(j * 16, 16)] = zeros16
        return _

    with jax.named_scope("zero_acc"):
        lax.fori_loop(0, CHUNK, _zero_row, None)
        r0 = sid * ROW_STRIDE
        for part in range(ROW_SPAN // CHUNK):
            pltpu.sync_copy(msg_a.at[pl.ds(0, CHUNK)],
                            acc_sh.at[pl.ds(r0 + part * CHUNK, CHUNK)])
        plsc.subcore_barrier()

    def _meta_start(g, b):
        pltpu.async_copy(col_hbm.at[g], cols[b], sem_m[b])
        pltpu.async_copy(row_hbm.at[g], rows[b], sem_m[b])
        pltpu.async_copy(val_hbm.at[g], vals[b], sem_m[b])

    def _meta_wait(g, b):
        pltpu.make_async_copy(col_hbm.at[g], cols[b], sem_m[b]).wait()
        pltpu.make_async_copy(row_hbm.at[g], rows[b], sem_m[b]).wait()
        pltpu.make_async_copy(val_hbm.at[g], vals[b], sem_m[b]).wait()

    def _scale_chunk(val_s, j, msg):
        def _scale(g, __):
            val16 = val_s[j, pl.ds(g * 16, 16)]
            for k in range(16):
                vb = _lane_bcast(val16, k)
                e = g * 16 + k
                for jj in range(8):
                    sl = pl.ds(jj * 16, 16)
                    msg[e, sl] = msg[e, sl] * vb
            return __

        lax.fori_loop(0, CHUNK // 16, _scale, None)

    def _run_super(tbl, b):
        col_s, row_s, val_s = cols[b], rows[b], vals[b]
        pltpu.async_copy(tbl.at[col_s.at[0]], msg_a, sem_a)
        pltpu.async_copy(tbl.at[col_s.at[1]], msg_b, sem_b)

        def _chunk_pair(t, _):
            j0 = 2 * t

            pltpu.make_async_copy(tbl.at[col_s.at[j0]], msg_a,
                                  sem_a).wait()
            _scale_chunk(val_s, j0, msg_a)
            pltpu.sync_copy(msg_a, acc_sh.at[row_s.at[j0]], add=True)

            @pl.when(t < SUPER // 2 - 1)
            def _():
                pltpu.async_copy(tbl.at[col_s.at[j0 + 2]], msg_a,
                                 sem_a)

            pltpu.make_async_copy(tbl.at[col_s.at[j0 + 1]], msg_b,
                                  sem_b).wait()
            _scale_chunk(val_s, j0 + 1, msg_b)
            pltpu.sync_copy(msg_b, acc_sh.at[row_s.at[j0 + 1]],
                            add=True)

            @pl.when(t < SUPER // 2 - 1)
            def _():
                pltpu.async_copy(tbl.at[col_s.at[j0 + 3]], msg_b,
                                 sem_b)

            return _

        lax.fori_loop(0, SUPER // 2, _chunk_pair, None)

    def _run_supers(tbl, base_g, npairs):
        @pl.when(npairs > 0)
        def _():
            _meta_start(base_g, 0)
            _meta_start(base_g + 1, 1)
            _meta_wait(base_g, 0)

            def _super_pair(t, _):
                g0 = base_g + 2 * t
                _run_super(tbl, 0)

                @pl.when(t < npairs - 1)
                def _():
                    _meta_start(g0 + 2, 0)

                _meta_wait(g0 + 1, 1)
                _run_super(tbl, 1)

                @pl.when(t < npairs - 1)
                def _():
                    _meta_start(g0 + 3, 1)
                    _meta_wait(g0 + 2, 0)

                return _

            lax.fori_loop(0, npairs, _super_pair, None)

    with jax.named_scope("edge_loop"):
        @pl.when(cid == 0)
        def _():
            _run_supers(h_hbm, sid * K0, K0 // 2)

        @pl.when(cid == 1)
        def _():
            _run_supers(hc_hbm, NS * K0 + sid * K1, K1 // 2)
        plsc.subcore_barrier()

    with jax.named_scope("copy_out"):
        pltpu.sync_copy(acc_sh.at[pl.ds(r0, ROW_SPAN)],
                        out_hbm.at[cid, pl.ds(r0, ROW_SPAN)])


_spmm = pl.kernel(
    _spmm_body,
    out_type=jax.ShapeDtypeStruct((NC, N, H), jnp.float32),
    mesh=plsc.VectorSubcoreMesh(core_axis_name="c", subcore_axis_name="s"),
    scratch_types=[
        pltpu.VMEM((SUPER, CHUNK), jnp.int32),
        pltpu.VMEM((SUPER, CHUNK), jnp.int32),
        pltpu.VMEM((SUPER, CHUNK), jnp.int32),
        pltpu.VMEM((SUPER, CHUNK), jnp.int32),
        pltpu.VMEM((SUPER, CHUNK), jnp.float32),
        pltpu.VMEM((SUPER, CHUNK), jnp.float32),
        pltpu.VMEM((CHUNK, H), jnp.float32),
        pltpu.VMEM((CHUNK, H), jnp.float32),
        pltpu.VMEM_SHARED((N, H), jnp.float32),
        pltpu.SemaphoreType.DMA,
        pltpu.SemaphoreType.DMA,
        pltpu.SemaphoreType.DMA,
        pltpu.SemaphoreType.DMA,
    ],
)


_GATHER_DNUMS = lax.GatherDimensionNumbers(
    offset_dims=(), collapsed_slice_dims=(0,), start_index_map=(0,))


def _lane_bcast(v16, k):
    return lax.gather(v16, jnp.full((16, 1), k, jnp.int32), _GATHER_DNUMS,
                      slice_sizes=(1,),
                      mode=lax.GatherScatterMode.PROMISE_IN_BOUNDS)


_SQRT_HALF = 0.7071067811865476


def _gelu(s):
    return 0.5 * s * (1.0 + lax.erf(s * _SQRT_HALF))


def _linear1_body(x_ref, w_ref, b_ref, o_ref):
    o_ref[...] = (
        jnp.dot(x_ref[...], w_ref[...], preferred_element_type=jnp.float32,
                precision=lax.Precision.HIGHEST)
        + b_ref[...]
    )


def _mid_body(p0_ref, p1_ref, w_ref, b_ref, o_ref):
    g = _gelu(p0_ref[...] + p1_ref[...])
    o_ref[...] = (
        jnp.dot(g, w_ref[...], preferred_element_type=jnp.float32,
                precision=lax.Precision.HIGHEST)
        + b_ref[...]
    )


def _final_body(p0_ref, p1_ref, x_ref, g_ref, b_ref, o_ref):
    h = _gelu(p0_ref[...] + p1_ref[...]) + x_ref[...]
    mean = jnp.mean(h, axis=-1, keepdims=True)
    var = jnp.mean((h - mean) ** 2, axis=-1, keepdims=True)
    o_ref[...] = (h - mean) * lax.rsqrt(var + 1e-5) * g_ref[...] + b_ref[...]


_BLK = 1000
_GRID = N // _BLK

_row_spec = pl.BlockSpec((_BLK, H), lambda i: (i, 0))
_full_spec = pl.BlockSpec((H, H), lambda i: (0, 0))
_vec_spec = pl.BlockSpec((1, H), lambda i: (0, 0))

_linear1 = pl.pallas_call(
    _linear1_body,
    grid=(_GRID,),
    in_specs=[_row_spec, _full_spec, _vec_spec],
    out_specs=_row_spec,
    out_shape=jax.ShapeDtypeStruct((N, H), jnp.float32),
)

_mid = pl.pallas_call(
    _mid_body,
    grid=(_GRID,),
    in_specs=[_row_spec, _row_spec, _full_spec, _vec_spec],
    out_specs=_row_spec,
    out_shape=jax.ShapeDtypeStruct((N, H), jnp.float32),
)

_final = pl.pallas_call(
    _final_body,
    grid=(_GRID,),
    in_specs=[_row_spec, _row_spec, _row_spec, _vec_spec, _vec_spec],
    out_specs=_row_spec,
    out_shape=jax.ShapeDtypeStruct((N, H), jnp.float32),
)


@jax.jit
def kernel(x, adj_indices, adj_values, W1, b1, W2, b2, ln_gamma, ln_beta):
    pad = EPAD - E
    row = jnp.concatenate([adj_indices[0], jnp.zeros((pad,), jnp.int32)])
    col = jnp.concatenate([adj_indices[1], jnp.zeros((pad,), jnp.int32)])
    val = jnp.concatenate([adj_values, jnp.zeros((pad,), jnp.float32)])
    row = row.reshape(TOTAL_SUPER, SUPER, CHUNK)
    col = col.reshape(TOTAL_SUPER, SUPER, CHUNK)
    val = val.reshape(TOTAL_SUPER, SUPER, CHUNK)

    b1r = b1.reshape(1, H)
    b2r = b2.reshape(1, H)
    gr = ln_gamma.reshape(1, H)
    br = ln_beta.reshape(1, H)

    h1 = _linear1(x, W1.T, b1r)
    h1c = lax.optimization_barrier(h1 * 1.0)
    p1 = _spmm(h1, h1c, row, col, val)
    h2 = _mid(p1[0], p1[1], W2.T, b2r)
    h2c = lax.optimization_barrier(h2 * 1.0)
    p2 = _spmm(h2, h2c, row, col, val)
    return _final(p2[0], p2[1], x[:, :H], gr, br)
